# baseline (speedup 1.0000x reference)
import functools

import jax
import jax.numpy as jnp
from jax import lax
from jax.experimental import pallas as pl
from jax.experimental.pallas import tpu as pltpu
from jax.experimental.pallas import tpu_sc as plsc

VOCAB = 100000
EMBED_DIM = 128
ROWS, SEQ = 4096, 50

NUM_CORES = 2
NUM_SUBCORES = 16
NW = NUM_CORES * NUM_SUBCORES

K_CHUNKS = 1
CROWS = ROWS // K_CHUNKS
R_PER_W = CROWS // NW

NBUF = 8
LAG = 2
N_GROUPS = R_PER_W // NBUF


def _emb_kernel(table_hbm, tok_hbm, out_hbm, idx_v, rows_v, gsems, ssems):
    wid = lax.axis_index("s") * NUM_CORES + lax.axis_index("c")
    base = wid * R_PER_W
    pltpu.sync_copy(tok_hbm.at[pl.ds(base, R_PER_W)], idx_v)

    def gather(r, b):
        return pltpu.make_async_copy(
            table_hbm.at[idx_v.at[r]],
            rows_v.at[b],
            gsems.at[b],
        )

    def scatter(r, b):
        return pltpu.make_async_copy(
            rows_v.at[b],
            out_hbm.at[base + r],
            ssems.at[b],
        )

    for b in range(NBUF):
        gather(b, b).start()

    def step(r, b, rl, bl, do_lag):
        gather(r, b).wait()
        scatter(r, b).start()
        if do_lag:
            scatter(rl, bl).wait()
            gather(rl + NBUF, bl).start()

    for b in range(NBUF):
        step(b, b, b - LAG, (b - LAG) % NBUF, b >= LAG)

    def group_body(gi, carry):
        r0 = gi * NBUF
        for b in range(NBUF):
            step(r0 + b, b, r0 + b - LAG, (b - LAG) % NBUF, True)
        return carry

    lax.fori_loop(1, N_GROUPS - 1, group_body, 0)

    r0 = (N_GROUPS - 1) * NBUF
    for b in range(NBUF):
        r = r0 + b
        gather(r, b).wait()
        scatter(r, b).start()
        rl, bl = r - LAG, (b - LAG) % NBUF
        scatter(rl, bl).wait()
        if rl + NBUF < R_PER_W:
            gather(rl + NBUF, bl).start()
    for k in range(LAG):
        r = R_PER_W - LAG + k
        scatter(r, r % NBUF).wait()


def _chunk_lookup(table, tok_chunk):
    mesh = plsc.VectorSubcoreMesh(core_axis_name="c", subcore_axis_name="s")
    return pl.kernel(
        _emb_kernel,
        out_type=jax.ShapeDtypeStruct((CROWS, SEQ, EMBED_DIM), jnp.float32),
        mesh=mesh,
        scratch_types=[
            pltpu.VMEM((R_PER_W, SEQ), jnp.int32),
            pltpu.VMEM((NBUF, SEQ, EMBED_DIM), jnp.float32),
            pltpu.SemaphoreType.DMA((NBUF,)),
            pltpu.SemaphoreType.DMA((NBUF,)),
        ],
        compiler_params=pltpu.CompilerParams(use_tc_tiling_on_sc=False),
    )(table, tok_chunk)


@functools.partial(jax.jit)
def _embedding_lookup(table, tokens):
    return _chunk_lookup(table, tokens)


def kernel(tokens, token_embedding):
    return _embedding_lookup(token_embedding, tokens)


# speedup vs baseline: 1.0183x; 1.0183x over previous
import functools

import jax
import jax.numpy as jnp
from jax import lax
from jax.experimental import pallas as pl
from jax.experimental.pallas import tpu as pltpu
from jax.experimental.pallas import tpu_sc as plsc

VOCAB = 100000
EMBED_DIM = 128
ROWS, SEQ = 4096, 50

NUM_CORES = 2
NUM_SUBCORES = 16
NW = NUM_CORES * NUM_SUBCORES

K_CHUNKS = 4
CROWS = ROWS // K_CHUNKS
R_PER_W = CROWS // NW

NBUF = 8
LAG = 2
N_GROUPS = R_PER_W // NBUF


def _emb_kernel(table_hbm, tok_hbm, out_hbm, idx_v, rows_v, gsems, ssems):
    wid = lax.axis_index("s") * NUM_CORES + lax.axis_index("c")
    base = wid * R_PER_W
    pltpu.sync_copy(tok_hbm.at[pl.ds(base, R_PER_W)], idx_v)

    def gather(r, b):
        return pltpu.make_async_copy(
            table_hbm.at[idx_v.at[r]],
            rows_v.at[b],
            gsems.at[b],
        )

    def scatter(r, b):
        return pltpu.make_async_copy(
            rows_v.at[b],
            out_hbm.at[base + r],
            ssems.at[b],
        )

    for b in range(NBUF):
        gather(b, b).start()

    def step(r, b, rl, bl, do_lag):
        gather(r, b).wait()
        scatter(r, b).start()
        if do_lag:
            scatter(rl, bl).wait()
            gather(rl + NBUF, bl).start()

    for b in range(NBUF):
        step(b, b, b - LAG, (b - LAG) % NBUF, b >= LAG)

    def group_body(gi, carry):
        r0 = gi * NBUF
        for b in range(NBUF):
            step(r0 + b, b, r0 + b - LAG, (b - LAG) % NBUF, True)
        return carry

    lax.fori_loop(1, N_GROUPS - 1, group_body, 0)

    r0 = (N_GROUPS - 1) * NBUF
    for b in range(NBUF):
        r = r0 + b
        gather(r, b).wait()
        scatter(r, b).start()
        rl, bl = r - LAG, (b - LAG) % NBUF
        scatter(rl, bl).wait()
        if rl + NBUF < R_PER_W:
            gather(rl + NBUF, bl).start()
    for k in range(LAG):
        r = R_PER_W - LAG + k
        scatter(r, r % NBUF).wait()


def _chunk_lookup(table, tok_chunk):
    mesh = plsc.VectorSubcoreMesh(core_axis_name="c", subcore_axis_name="s")
    return pl.kernel(
        _emb_kernel,
        out_type=jax.ShapeDtypeStruct((CROWS, SEQ, EMBED_DIM), jnp.float32),
        mesh=mesh,
        scratch_types=[
            pltpu.VMEM((R_PER_W, SEQ), jnp.int32),
            pltpu.VMEM((NBUF, SEQ, EMBED_DIM), jnp.float32),
            pltpu.SemaphoreType.DMA((NBUF,)),
            pltpu.SemaphoreType.DMA((NBUF,)),
        ],
    )(table, tok_chunk)


@functools.partial(jax.jit)
def _embedding_lookup(table, tokens):
    acc = jnp.zeros((ROWS, SEQ, EMBED_DIM), jnp.float32)
    for c in range(K_CHUNKS):
        part = _chunk_lookup(
            table, lax.slice_in_dim(tokens, c * CROWS, (c + 1) * CROWS)
        )
        acc = lax.dynamic_update_slice(acc, part, (c * CROWS, 0, 0))
    return acc


def kernel(tokens, token_embedding):
    return _embedding_lookup(token_embedding, tokens)


# speedup vs baseline: 1.7854x; 1.7533x over previous
import functools

import jax
import jax.numpy as jnp
from jax import lax
from jax.experimental import pallas as pl
from jax.experimental.pallas import tpu as pltpu
from jax.experimental.pallas import tpu_sc as plsc

VOCAB = 100000
EMBED_DIM = 128
ROWS, SEQ = 4096, 50

NUM_CORES = 2
NUM_SUBCORES = 16
NW = NUM_CORES * NUM_SUBCORES

K_CHUNKS = 1
CROWS = ROWS // K_CHUNKS
R_PER_W = CROWS // NW

NBUF = 8
LAG = 4
N_GROUPS = R_PER_W // NBUF


def _emb_kernel(table_hbm, tok_hbm, out_hbm, idx_v, rows_v, gsems, ssems):
    wid = lax.axis_index("s") * NUM_CORES + lax.axis_index("c")
    base = wid * R_PER_W
    pltpu.sync_copy(tok_hbm.at[pl.ds(base, R_PER_W)], idx_v)

    def gather(r, b):
        return pltpu.make_async_copy(
            table_hbm.at[idx_v.at[r]],
            rows_v.at[b],
            gsems.at[b],
        )

    def scatter(r, b):
        return pltpu.make_async_copy(
            rows_v.at[b],
            out_hbm.at[base + r],
            ssems.at[b],
        )

    for b in range(NBUF):
        gather(b, b).start()

    def step(r, b, rl, bl, do_lag):
        gather(r, b).wait()
        scatter(r, b).start()
        if do_lag:
            scatter(rl, bl).wait()
            gather(rl + NBUF, bl).start()

    for b in range(NBUF):
        step(b, b, b - LAG, (b - LAG) % NBUF, b >= LAG)

    def group_body(gi, carry):
        r0 = gi * NBUF
        for b in range(NBUF):
            step(r0 + b, b, r0 + b - LAG, (b - LAG) % NBUF, True)
        return carry

    lax.fori_loop(1, N_GROUPS - 1, group_body, 0)

    r0 = (N_GROUPS - 1) * NBUF
    for b in range(NBUF):
        r = r0 + b
        gather(r, b).wait()
        scatter(r, b).start()
        rl, bl = r - LAG, (b - LAG) % NBUF
        scatter(rl, bl).wait()
        if rl + NBUF < R_PER_W:
            gather(rl + NBUF, bl).start()
    for k in range(LAG):
        r = R_PER_W - LAG + k
        scatter(r, r % NBUF).wait()


def _chunk_lookup(table, tok_chunk):
    mesh = plsc.VectorSubcoreMesh(core_axis_name="c", subcore_axis_name="s")
    return pl.kernel(
        _emb_kernel,
        out_type=jax.ShapeDtypeStruct((CROWS, SEQ, EMBED_DIM), jnp.float32),
        mesh=mesh,
        scratch_types=[
            pltpu.VMEM((R_PER_W, SEQ), jnp.int32),
            pltpu.VMEM((NBUF, SEQ, EMBED_DIM), jnp.float32),
            pltpu.SemaphoreType.DMA((NBUF,)),
            pltpu.SemaphoreType.DMA((NBUF,)),
        ],
    )(table, tok_chunk)


@functools.partial(jax.jit)
def _embedding_lookup(table, tokens):
    return _chunk_lookup(table, tokens)


def kernel(tokens, token_embedding):
    return _embedding_lookup(token_embedding, tokens)


# speedup vs baseline: 1.7999x; 1.0081x over previous
import functools

import jax
import jax.numpy as jnp
from jax import lax
from jax.experimental import pallas as pl
from jax.experimental.pallas import tpu as pltpu
from jax.experimental.pallas import tpu_sc as plsc

VOCAB = 100000
EMBED_DIM = 128
ROWS, SEQ = 4096, 50

NUM_CORES = 2
NUM_SUBCORES = 16
NW = NUM_CORES * NUM_SUBCORES

K_CHUNKS = 1
CROWS = ROWS // K_CHUNKS
R_PER_W = CROWS // NW

PAIR = 2
P_PER_W = R_PER_W // PAIR
NBUF = 8
LAG = 2
N_GROUPS = P_PER_W // NBUF


def _emb_kernel(table_hbm, tok_hbm, out_hbm, idx_v, rows_v, gsems, ssems):
    wid = lax.axis_index("s") * NUM_CORES + lax.axis_index("c")
    base = wid * R_PER_W
    pltpu.sync_copy(tok_hbm.at[pl.ds(base, R_PER_W)], idx_v)

    class _Pair:
        def __init__(self, copies):
            self.copies = copies

        def start(self):
            for c in self.copies:
                c.start()

        def wait(self):
            for c in self.copies:
                c.wait()

    def gather(p, b):
        return _Pair([
            pltpu.make_async_copy(
                table_hbm.at[idx_v.at[p * PAIR + k]],
                rows_v.at[b, k],
                gsems.at[b],
            )
            for k in range(PAIR)
        ])

    def scatter(p, b):
        return pltpu.make_async_copy(
            rows_v.at[b],
            out_hbm.at[pl.ds(base + p * PAIR, PAIR)],
            ssems.at[b],
        )

    for b in range(NBUF):
        gather(b, b).start()

    def step(r, b, rl, bl, do_lag):
        gather(r, b).wait()
        scatter(r, b).start()
        if do_lag:
            scatter(rl, bl).wait()
            gather(rl + NBUF, bl).start()

    for b in range(NBUF):
        step(b, b, b - LAG, (b - LAG) % NBUF, b >= LAG)

    def group_body(gi, carry):
        r0 = gi * NBUF
        for b in range(NBUF):
            step(r0 + b, b, r0 + b - LAG, (b - LAG) % NBUF, True)
        return carry

    lax.fori_loop(1, N_GROUPS - 1, group_body, 0)

    r0 = (N_GROUPS - 1) * NBUF
    for b in range(NBUF):
        r = r0 + b
        gather(r, b).wait()
        scatter(r, b).start()
        rl, bl = r - LAG, (b - LAG) % NBUF
        scatter(rl, bl).wait()
        if rl + NBUF < P_PER_W:
            gather(rl + NBUF, bl).start()
    for k in range(LAG):
        r = P_PER_W - LAG + k
        scatter(r, r % NBUF).wait()


def _chunk_lookup(table, tok_chunk):
    mesh = plsc.VectorSubcoreMesh(core_axis_name="c", subcore_axis_name="s")
    return pl.kernel(
        _emb_kernel,
        out_type=jax.ShapeDtypeStruct((CROWS, SEQ, EMBED_DIM), jnp.float32),
        mesh=mesh,
        scratch_types=[
            pltpu.VMEM((R_PER_W, SEQ), jnp.int32),
            pltpu.VMEM((NBUF, PAIR, SEQ, EMBED_DIM), jnp.float32),
            pltpu.SemaphoreType.DMA((NBUF,)),
            pltpu.SemaphoreType.DMA((NBUF,)),
        ],
    )(table, tok_chunk)


@functools.partial(jax.jit)
def _embedding_lookup(table, tokens):
    return _chunk_lookup(table, tokens)


def kernel(tokens, token_embedding):
    return _embedding_lookup(token_embedding, tokens)
